# baseline (speedup 1.0000x reference)
import functools

import jax
import jax.numpy as jnp
from jax import lax
from jax.experimental import pallas as pl
from jax.experimental.pallas import tpu as pltpu
from jax.experimental.pallas import tpu_sc as plsc

B = 16
N = 1000
HW = 576
TOPK = 25
NCHUNK = 63
ROWPAD = 1008

_BIG = 1 << 30
_NEG = float("-inf")


def _sc_loss(p_flat, logits_flat, feat2d):
    mesh = plsc.VectorSubcoreMesh(
        core_axis_name="c", subcore_axis_name="s", num_cores=1
    )

    @functools.partial(
        pl.kernel,
        out_type=jax.ShapeDtypeStruct((16,), jnp.float32),
        mesh=mesh,
        scratch_types=[
            pltpu.VMEM((ROWPAD,), jnp.float32),
            pltpu.VMEM((HW,), jnp.float32),
            pltpu.VMEM((TOPK,), jnp.int32),
            pltpu.VMEM((TOPK, HW), jnp.float32),
            pltpu.VMEM((16,), jnp.float32),
            pltpu.VMEM((B * 16,), jnp.float32),
            pltpu.VMEM_SHARED((B * 16,), jnp.float32),
            pltpu.SemaphoreType.DMA,
        ],
        compiler_params=pltpu.CompilerParams(
            needs_layout_passes=False, use_tc_tiling_on_sc=False
        ),
    )
    def k(p_hbm, lg_hbm, ft_hbm, out_hbm,
          row_v, pbuf_v, gidx_v, rows_v, part_v, sum_v, shared, sem):
        b = lax.axis_index("s")
        lane = lax.broadcasted_iota(jnp.int32, (16,), 0)
        lane0 = lane == 0

        pltpu.sync_copy(lg_hbm.at[pl.ds(b * N, N)], row_v.at[pl.ds(0, N)])
        pltpu.sync_copy(p_hbm.at[pl.ds(b * HW, HW)], pbuf_v)

        tail = row_v[pl.ds(ROWPAD - 16, 16)]
        row_v[pl.ds(ROWPAD - 16, 16)] = jnp.where(lane < 8, tail, _NEG)

        def extract(kk, carry):
            def scan(j, acc):
                acc_v, acc_i = acc
                v = row_v[pl.ds(j * 16, 16)]
                i = lane + j * 16
                take = v > acc_v
                return (jnp.where(take, v, acc_v), jnp.where(take, i, acc_i))

            acc_v, acc_i = lax.fori_loop(
                0, NCHUNK, scan,
                (jnp.full((16,), _NEG), jnp.full((16,), _BIG)),
                unroll=7,
            )
            m = jnp.max(acc_v)
            ci = jnp.min(jnp.where(acc_v == m, acc_i, _BIG))
            plsc.store_scatter(
                gidx_v, [jnp.full((16,), kk, jnp.int32)],
                jnp.full((16,), b * N + ci, jnp.int32), mask=lane0)
            plsc.store_scatter(
                row_v, [jnp.full((16,), ci, jnp.int32)],
                jnp.full((16,), _NEG), mask=lane0)
            return carry

        lax.fori_loop(0, TOPK, extract, jnp.int32(0))

        cp = pltpu.make_async_copy(ft_hbm.at[gidx_v], rows_v, sem)
        cp.start()

        def psum_step(t, acc):
            return acc + pbuf_v[pl.ds(t * 16, 16)]

        acc = lax.fori_loop(0, HW // 16, psum_step, jnp.zeros((16,), jnp.float32),
                            unroll=6)

        cp.wait()

        def frow(i, acc):
            for c in range(HW // 16):
                acc = acc + rows_v[i, pl.ds(c * 16, 16)]
            return acc

        acc = lax.fori_loop(0, TOPK, frow, acc)

        part_v[...] = acc
        pltpu.sync_copy(part_v, shared.at[pl.ds(b * 16, 16)])
        plsc.subcore_barrier()

        @pl.when(b == 0)
        def _():
            pltpu.sync_copy(shared, sum_v)

            def tot_step(i, t):
                return t + sum_v[pl.ds(i * 16, 16)]

            tot = lax.fori_loop(0, B, tot_step, jnp.zeros((16,), jnp.float32))
            total = jnp.sum(tot) * jnp.float32(1.0 / (B * HW))
            part_v[...] = jnp.full((16,), total)
            pltpu.sync_copy(part_v, out_hbm)

    return k(p_flat, logits_flat, feat2d)


@jax.jit
def kernel(p, main_out, features):
    p_flat = p.reshape(-1)
    logits_flat = main_out.reshape(-1)
    feat2d = features.reshape(B * N, HW)
    out = _sc_loss(p_flat, logits_flat, feat2d)
    return out[0]
